# Initial kernel scaffold; baseline (speedup 1.0000x reference)
import jax
import jax.numpy as jnp
from jax.experimental import pallas as pl


def kernel(x, edge_index, W, b):
    raise NotImplementedError("write your pallas kernel here")


# speedup vs baseline: 6.9726x; 6.9726x over previous
import functools

import jax
import jax.numpy as jnp
from jax import lax
from jax.experimental import pallas as pl
from jax.experimental.pallas import tpu as pltpu
from jax.experimental.pallas import tpu_sc as plsc

N = 10000
E = 320000
D_IN = 128
D_OUT = 256

NC = 2
NS = 16
NW = NC * NS
EPW = E // NW
CH = 80
NCHUNK = EPW // CH
RPT = 624
ZB = 16


def _hist_body(idx_hbm, out_hbm, src_v, dst_v, hs_v, hd_v):
    cid = lax.axis_index("c")
    sid = lax.axis_index("s")
    wid = sid * NC + cid

    pltpu.sync_copy(idx_hbm.at[wid, 0], src_v)
    pltpu.sync_copy(idx_hbm.at[wid, 1], dst_v)

    zeros = jnp.zeros((16,), jnp.float32)

    def zero_body(i, _):
        hs_v[pl.ds(i * 16, 16)] = zeros
        hd_v[pl.ds(i * 16, 16)] = zeros
        return 0

    lax.fori_loop(0, N // 16, zero_body, 0)

    ones = jnp.ones((16,), jnp.float32)

    def hist_body(i, _):
        s_idx = src_v[pl.ds(i * 16, 16)]
        d_idx = dst_v[pl.ds(i * 16, 16)]
        plsc.addupdate_scatter(hs_v, [s_idx], ones)
        plsc.addupdate_scatter(hd_v, [d_idx], ones)
        return 0

    lax.fori_loop(0, EPW // 16, hist_body, 0)

    pltpu.sync_copy(hs_v, out_hbm.at[wid, 0])
    pltpu.sync_copy(hd_v, out_hbm.at[wid, 1])


_hist = pl.kernel(
    _hist_body,
    out_type=jax.ShapeDtypeStruct((NW, 2, N), jnp.float32),
    compiler_params=pltpu.CompilerParams(needs_layout_passes=False),
    mesh=plsc.VectorSubcoreMesh(core_axis_name="c", subcore_axis_name="s"),
    scratch_types=[
        pltpu.VMEM((EPW,), jnp.int32),
        pltpu.VMEM((EPW,), jnp.int32),
        pltpu.VMEM((N,), jnp.float32),
        pltpu.VMEM((N,), jnp.float32),
    ],
)


def _scale_body(x_ref, cnt_ref, o_ref):
    deg = jnp.sum(cnt_ref[...], axis=1, keepdims=True)
    scale = lax.rsqrt(jnp.maximum(deg, 1.0))
    o_ref[...] = x_ref[...] * scale


_BLK = 2000


def _scale_x(x, cnt_t):
    return pl.pallas_call(
        _scale_body,
        grid=(N // _BLK,),
        in_specs=[
            pl.BlockSpec((_BLK, D_IN), lambda i: (i, 0)),
            pl.BlockSpec((_BLK, NW), lambda i: (i, 0)),
        ],
        out_specs=pl.BlockSpec((_BLK, D_IN), lambda i: (i, 0)),
        out_shape=jax.ShapeDtypeStruct((N, D_IN), jnp.float32),
    )(x, cnt_t)


def _agg_body(xn_hbm, src_hbm, dst_hbm, out_hbm, src_v, dst_v, rows_v, zb_v,
              agg_s, sem):
    cid = lax.axis_index("c")
    sid = lax.axis_index("s")
    wid = sid * NC + cid

    pltpu.sync_copy(src_hbm.at[wid], src_v)
    pltpu.sync_copy(dst_hbm.at[wid], dst_v)

    zeros = jnp.zeros((16,), jnp.float32)
    for i in range(ZB):
        for j in range(D_IN // 16):
            zb_v[i, pl.ds(j * 16, 16)] = zeros

    def zagg(k, _):
        pltpu.sync_copy(zb_v, agg_s.at[pl.ds(sid * RPT + k * ZB, ZB)])
        return 0

    lax.fori_loop(0, RPT // ZB, zagg, 0)

    @pl.when(sid == NS - 1)
    def _():
        pltpu.sync_copy(zb_v, agg_s.at[pl.ds(NS * RPT, ZB)])

    plsc.subcore_barrier()

    def chunk_body(c, _):
        pltpu.async_copy(xn_hbm.at[src_v.at[c]], rows_v, sem).wait()
        pltpu.sync_copy(rows_v, agg_s.at[dst_v.at[c]], add=True)
        return 0

    lax.fori_loop(0, NCHUNK, chunk_body, 0)
    plsc.subcore_barrier()

    r0 = sid * RPT
    pltpu.sync_copy(agg_s.at[pl.ds(r0, RPT)],
                    out_hbm.at[cid, pl.ds(r0, RPT)])

    @pl.when(sid == NS - 1)
    def _():
        pltpu.sync_copy(agg_s.at[pl.ds(NS * RPT, ZB)],
                        out_hbm.at[cid, pl.ds(NS * RPT, ZB)])


_agg = pl.kernel(
    _agg_body,
    out_type=jax.ShapeDtypeStruct((NC, N, D_IN), jnp.float32),
    compiler_params=pltpu.CompilerParams(needs_layout_passes=False),
    mesh=plsc.VectorSubcoreMesh(core_axis_name="c", subcore_axis_name="s"),
    scratch_types=[
        pltpu.VMEM((NCHUNK, CH), jnp.int32),
        pltpu.VMEM((NCHUNK, CH), jnp.int32),
        pltpu.VMEM((CH, D_IN), jnp.float32),
        pltpu.VMEM((ZB, D_IN), jnp.float32),
        pltpu.VMEM_SHARED((N, D_IN), jnp.float32),
        pltpu.SemaphoreType.DMA,
    ],
)


def _final_body(parts_ref, cnt_ref, w_ref, b_ref, o_ref):
    agg = parts_ref[0, :, :] + parts_ref[1, :, :]
    deg = jnp.sum(cnt_ref[...], axis=1, keepdims=True)
    scale = lax.rsqrt(jnp.maximum(deg, 1.0))
    acc = jnp.dot(agg * scale, w_ref[...], preferred_element_type=jnp.float32)
    o_ref[...] = jnp.maximum(acc + b_ref[...], 0.0)


def _final(parts, cnt_t, W, b2):
    return pl.pallas_call(
        _final_body,
        grid=(N // _BLK,),
        in_specs=[
            pl.BlockSpec((NC, _BLK, D_IN), lambda i: (0, i, 0)),
            pl.BlockSpec((_BLK, NW), lambda i: (i, 0)),
            pl.BlockSpec((D_IN, D_OUT), lambda i: (0, 0)),
            pl.BlockSpec((1, D_OUT), lambda i: (0, 0)),
        ],
        out_specs=pl.BlockSpec((_BLK, D_OUT), lambda i: (i, 0)),
        out_shape=jax.ShapeDtypeStruct((N, D_OUT), jnp.float32),
    )(parts, cnt_t, W, b2)


@jax.jit
def kernel(x, edge_index, W, b):
    idx_w = edge_index.reshape(2, NW, EPW).transpose(1, 0, 2)
    counts = _hist(idx_w)
    cnt_src_t = counts[:, 0, :].T
    cnt_dst_t = counts[:, 1, :].T
    x_norm = _scale_x(x, cnt_src_t)
    src_rs = edge_index[0].reshape(NW, NCHUNK, CH)
    dst_rs = edge_index[1].reshape(NW, NCHUNK, CH)
    parts = _agg(x_norm, src_rs, dst_rs)
    return _final(parts, cnt_dst_t, W, b.reshape(1, D_OUT))
